# baseline (speedup 1.0000x reference)
import functools

import jax
import jax.numpy as jnp
from jax import lax
from jax.experimental import pallas as pl
from jax.experimental.pallas import tpu as pltpu
from jax.experimental.pallas import tpu_sc as plsc

_ROWS = 8192
_COLS = 1024
_NW = 32
_ROWS_PER_W = _ROWS // _NW
_CHUNK = 64
_N_CHUNKS = _ROWS_PER_W // _CHUNK


@functools.partial(
    pl.kernel,
    out_type=jax.ShapeDtypeStruct((_ROWS, _COLS), jnp.float32),
    mesh=plsc.VectorSubcoreMesh(core_axis_name="c", subcore_axis_name="s"),
    scratch_types=[pltpu.VMEM((_CHUNK, _COLS), jnp.float32)],
)
def _sc_copy(table_hbm, out_hbm, buf):
    wid = lax.axis_index("s") * 2 + lax.axis_index("c")
    base = wid * _ROWS_PER_W
    for j in range(_N_CHUNKS):
        r = base + j * _CHUNK
        pltpu.sync_copy(table_hbm.at[pl.ds(r, _CHUNK)], buf)
        pltpu.sync_copy(buf, out_hbm.at[pl.ds(r, _CHUNK)])


def kernel(inputs, pos_table):
    del inputs
    return _sc_copy(pos_table)
